# baseline (speedup 1.0000x reference)
import jax
import jax.numpy as jnp
from jax import lax
from jax.experimental import pallas as pl
from jax.experimental.pallas import tpu as pltpu
from jax.experimental.pallas import tpu_sc as plsc

N_TOKENS = 16384
D = 1000

_info = plsc.get_sparse_core_info()
_NC = _info.num_cores
_NS = _info.num_subcores
_NW = _NC * _NS
_B_PER_W = N_TOKENS // _NW
_CHUNK = 64
_N_CHUNKS = _B_PER_W // _CHUNK


def _gather_body(idx_hbm, table_hbm, out_hbm, idx_v, rows_v, sem0, sem1):
    c = lax.axis_index("c")
    s = lax.axis_index("s")
    wid = s * _NC + c
    pltpu.sync_copy(idx_hbm.at[wid], idx_v)
    base = wid * _B_PER_W
    sems = (sem0, sem1)
    copies = [None, None]
    copies[0] = pltpu.async_copy(table_hbm.at[idx_v.at[0]], rows_v.at[0], sems[0])
    for g in range(_N_CHUNKS):
        buf = g % 2
        nxt = (g + 1) % 2
        if g + 1 < _N_CHUNKS:
            copies[nxt] = pltpu.async_copy(
                table_hbm.at[idx_v.at[g + 1]], rows_v.at[nxt], sems[nxt])
        copies[buf].wait()
        pltpu.sync_copy(rows_v.at[buf],
                        out_hbm.at[pl.ds(base + g * _CHUNK, _CHUNK)])


def kernel(x, table):
    idx = x.astype(jnp.int32).reshape(_NW, _N_CHUNKS, _CHUNK)
    gather = pl.kernel(
        _gather_body,
        out_type=jax.ShapeDtypeStruct((N_TOKENS, D), jnp.float32),
        mesh=plsc.VectorSubcoreMesh(core_axis_name="c", subcore_axis_name="s"),
        scratch_types=[
            pltpu.VMEM((_N_CHUNKS, _CHUNK), jnp.int32),
            pltpu.VMEM((2, _CHUNK, D), jnp.float32),
            pltpu.SemaphoreType.DMA,
            pltpu.SemaphoreType.DMA,
        ],
        compiler_params=pltpu.CompilerParams(use_tc_tiling_on_sc=False),
    )
    return gather(idx, table)


# speedup vs baseline: 1.0863x; 1.0863x over previous
import jax
import jax.numpy as jnp
from jax import lax
from jax.experimental import pallas as pl
from jax.experimental.pallas import tpu as pltpu
from jax.experimental.pallas import tpu_sc as plsc

N_TOKENS = 16384
D = 1000
VOCAB_PAD = 1024

_info = plsc.get_sparse_core_info()
_NC = _info.num_cores
_NS = _info.num_subcores
_NW = _NC * _NS
_B_PER_W = N_TOKENS // _NW
_CHUNK = 16
_N_CHUNKS = _B_PER_W // _CHUNK
_NBUF = 4
_ROWS_PER_TILE = VOCAB_PAD // _NS


def _gather_body(idx_hbm, table_hbm, out_hbm, tab_sh, idx_v, rows_v,
                 gsems, wsems):
    c = lax.axis_index("c")
    s = lax.axis_index("s")
    wid = s * _NC + c
    base = wid * _B_PER_W

    pltpu.sync_copy(idx_hbm.at[wid], idx_v)
    pltpu.sync_copy(table_hbm.at[pl.ds(s * _ROWS_PER_TILE, _ROWS_PER_TILE)],
                    tab_sh.at[pl.ds(s * _ROWS_PER_TILE, _ROWS_PER_TILE)])
    plsc.subcore_barrier()

    gathers = [None] * _NBUF
    writes = [None] * _NBUF
    for b in range(_NBUF):
        gathers[b] = pltpu.async_copy(
            tab_sh.at[idx_v.at[b]], rows_v.at[b], gsems.at[b])
    for g in range(_N_CHUNKS):
        buf = g % _NBUF
        gathers[buf].wait()
        writes[buf] = pltpu.async_copy(
            rows_v.at[buf], out_hbm.at[pl.ds(base + g * _CHUNK, _CHUNK)],
            wsems.at[buf])
        if g + _NBUF < _N_CHUNKS:
            writes[buf].wait()
            gathers[buf] = pltpu.async_copy(
                tab_sh.at[idx_v.at[g + _NBUF]], rows_v.at[buf], gsems.at[buf])
    for g in range(_N_CHUNKS - _NBUF, _N_CHUNKS):
        writes[g % _NBUF].wait()


def kernel(x, table):
    idx = x.astype(jnp.int32).reshape(_NW, _N_CHUNKS, _CHUNK)
    table_p = jnp.pad(table, ((0, VOCAB_PAD - table.shape[0]), (0, 0)))
    gather = pl.kernel(
        _gather_body,
        out_type=jax.ShapeDtypeStruct((N_TOKENS, D), jnp.float32),
        mesh=plsc.VectorSubcoreMesh(core_axis_name="c", subcore_axis_name="s"),
        scratch_types=[
            pltpu.VMEM_SHARED((VOCAB_PAD, D), jnp.float32),
            pltpu.VMEM((_N_CHUNKS, _CHUNK), jnp.int32),
            pltpu.VMEM((_NBUF, _CHUNK, D), jnp.float32),
            pltpu.SemaphoreType.DMA((_NBUF,)),
            pltpu.SemaphoreType.DMA((_NBUF,)),
        ],
        compiler_params=pltpu.CompilerParams(use_tc_tiling_on_sc=False),
    )
    return gather(idx, table_p)


# speedup vs baseline: 1.0890x; 1.0025x over previous
import jax
import jax.numpy as jnp
from jax import lax
from jax.experimental import pallas as pl
from jax.experimental.pallas import tpu as pltpu
from jax.experimental.pallas import tpu_sc as plsc

N_TOKENS = 16384
D = 1000
VOCAB = 1000

_info = plsc.get_sparse_core_info()
_NC = _info.num_cores
_NS = _info.num_subcores
_NW = _NC * _NS
_B_PER_W = N_TOKENS // _NW
_CHUNK = 16
_N_CHUNKS = _B_PER_W // _CHUNK
_NBUF = 4
_STRIPE = 63
_LAST = VOCAB - 15 * _STRIPE


def _gather_body(idx_hbm, table_hbm, out_hbm, tab_sh, idx_v, rows_v,
                 gsems, wsems):
    c = lax.axis_index("c")
    s = lax.axis_index("s")
    wid = s * _NC + c
    base = wid * _B_PER_W

    pltpu.sync_copy(idx_hbm.at[wid], idx_v)
    @pl.when(s < _NS - 1)
    def _():
        pltpu.sync_copy(table_hbm.at[pl.ds(s * _STRIPE, _STRIPE)],
                        tab_sh.at[pl.ds(s * _STRIPE, _STRIPE)])

    @pl.when(s == _NS - 1)
    def _():
        pltpu.sync_copy(table_hbm.at[pl.ds((_NS - 1) * _STRIPE, _LAST)],
                        tab_sh.at[pl.ds((_NS - 1) * _STRIPE, _LAST)])

    plsc.subcore_barrier()

    gathers = [None] * _NBUF
    writes = [None] * _NBUF
    for b in range(_NBUF):
        gathers[b] = pltpu.async_copy(
            tab_sh.at[idx_v.at[b]], rows_v.at[b], gsems.at[b])
    for g in range(_N_CHUNKS):
        buf = g % _NBUF
        gathers[buf].wait()
        writes[buf] = pltpu.async_copy(
            rows_v.at[buf], out_hbm.at[pl.ds(base + g * _CHUNK, _CHUNK)],
            wsems.at[buf])
        if g + _NBUF < _N_CHUNKS:
            writes[buf].wait()
            gathers[buf] = pltpu.async_copy(
                tab_sh.at[idx_v.at[g + _NBUF]], rows_v.at[buf], gsems.at[buf])
    for g in range(_N_CHUNKS - _NBUF, _N_CHUNKS):
        writes[g % _NBUF].wait()


def kernel(x, table):
    idx = x.astype(jnp.int32).reshape(_NW, _N_CHUNKS, _CHUNK)
    gather = pl.kernel(
        _gather_body,
        out_type=jax.ShapeDtypeStruct((N_TOKENS, D), jnp.float32),
        mesh=plsc.VectorSubcoreMesh(core_axis_name="c", subcore_axis_name="s"),
        scratch_types=[
            pltpu.VMEM_SHARED((VOCAB, D), jnp.float32),
            pltpu.VMEM((_N_CHUNKS, _CHUNK), jnp.int32),
            pltpu.VMEM((_NBUF, _CHUNK, D), jnp.float32),
            pltpu.SemaphoreType.DMA((_NBUF,)),
            pltpu.SemaphoreType.DMA((_NBUF,)),
        ],
        compiler_params=pltpu.CompilerParams(use_tc_tiling_on_sc=False),
    )
    return gather(idx, table)


# speedup vs baseline: 1.3993x; 1.2850x over previous
import jax
import jax.numpy as jnp
from jax import lax
from jax.experimental import pallas as pl
from jax.experimental.pallas import tpu as pltpu
from jax.experimental.pallas import tpu_sc as plsc

N_TOKENS = 16384
D = 1000
VOCAB = 1000
D_MAIN = 896
D_TAIL = D - D_MAIN
_LANES = 16

_info = plsc.get_sparse_core_info()
_NC = _info.num_cores
_NS = _info.num_subcores
_NW = _NC * _NS
_B_PER_W = N_TOKENS // _NW
_CHUNK = 16
_N_CHUNKS = _B_PER_W // _CHUNK
_NBUF = 6


def _gather_body(idx_hbm, tab_main_hbm, tab_tail_hbm, out_hbm,
                 idx_v, rows_v, tail_v, gsems, tsems, wsems):
    c = lax.axis_index("c")
    s = lax.axis_index("s")
    wid = s * _NC + c
    base = wid * _B_PER_W

    pltpu.sync_copy(idx_hbm.at[wid], idx_v)

    def start_gathers(g, buf):
        ga = pltpu.async_copy(tab_main_hbm.at[idx_v.at[g]],
                              rows_v.at[buf, :, pl.ds(0, D_MAIN)],
                              gsems.at[buf])
        gt = pltpu.async_copy(tab_tail_hbm.at[idx_v.at[g]],
                              tail_v.at[buf], tsems.at[buf])
        return ga, gt

    def patch_tail(buf):
        lanes = jnp.arange(_LANES, dtype=jnp.int32)

        def row_body(r, carry):
            for k in range(6):
                off = k * _LANES
                v = tail_v[buf, r, pl.ds(off, _LANES)]
                rows_v[buf, r, pl.ds(D_MAIN + off, _LANES)] = v
            v = tail_v[buf, r, pl.ds(6 * _LANES, _LANES)]
            plsc.store_scatter(
                rows_v,
                [jnp.full((_LANES,), buf, jnp.int32),
                 jnp.full((_LANES,), r, jnp.int32),
                 (D_MAIN + 6 * _LANES) + lanes],
                v, mask=lanes < (D - D_MAIN - 6 * _LANES))
            return carry
        lax.fori_loop(0, _CHUNK, row_body, 0)

    gathers = [None] * _NBUF
    writes = [None] * _NBUF
    for b in range(_NBUF):
        gathers[b] = start_gathers(b, b)
    for g in range(_N_CHUNKS):
        buf = g % _NBUF
        ga, gt = gathers[buf]
        gt.wait()
        ga.wait()
        patch_tail(buf)
        writes[buf] = pltpu.async_copy(
            rows_v.at[buf],
            out_hbm.at[pl.ds(base + g * _CHUNK, _CHUNK)],
            wsems.at[buf])
        if g + _NBUF < _N_CHUNKS:
            writes[buf].wait()
            gathers[buf] = start_gathers(g + _NBUF, buf)
    for g in range(_N_CHUNKS - _NBUF, _N_CHUNKS):
        writes[g % _NBUF].wait()


def kernel(x, table):
    idx = x.astype(jnp.int32).reshape(_NW, _N_CHUNKS, _CHUNK)
    tab_main = table[:, :D_MAIN]
    tab_tail = jnp.pad(table[:, D_MAIN:], ((0, 0), (0, 128 - D_TAIL)))
    gather = pl.kernel(
        _gather_body,
        out_type=jax.ShapeDtypeStruct((N_TOKENS, D), jnp.float32),
        mesh=plsc.VectorSubcoreMesh(core_axis_name="c", subcore_axis_name="s"),
        scratch_types=[
            pltpu.VMEM((_N_CHUNKS, _CHUNK), jnp.int32),
            pltpu.VMEM((_NBUF, _CHUNK, D), jnp.float32),
            pltpu.VMEM((_NBUF, _CHUNK, 128), jnp.float32),
            pltpu.SemaphoreType.DMA((_NBUF,)),
            pltpu.SemaphoreType.DMA((_NBUF,)),
            pltpu.SemaphoreType.DMA((_NBUF,)),
        ],
        compiler_params=pltpu.CompilerParams(needs_layout_passes=False),
    )
    return gather(idx, tab_main, tab_tail)
